# baseline (speedup 1.0000x reference)
import functools

import jax
import jax.numpy as jnp
from jax import lax
from jax.experimental import pallas as pl
from jax.experimental.pallas import tpu as pltpu
from jax.experimental.pallas import tpu_sc as plsc

B = 16384
V = 1000000
D = 16
N_CAT = 10
D_IN = N_CAT * D + 3
H = 32

NC = 2
NS = 16
NW = NC * NS
TOT = B * N_CAT
PER_W = TOT // NW
ROWS_W = B // NW
CHUNK = 128
NCH = PER_W // CHUNK
NBUF = 2
L = 16


def _sc_gather(table2, idx3, bl_tab, col_tab):
    mesh = plsc.VectorSubcoreMesh(core_axis_name="c", subcore_axis_name="s")

    @functools.partial(
        pl.kernel,
        out_type=jax.ShapeDtypeStruct((B, N_CAT * D), jnp.float32),
        mesh=mesh,
        scratch_types=[
            pltpu.VMEM((NCH, CHUNK), jnp.int32),
            pltpu.VMEM((NCH, CHUNK), jnp.int32),
            pltpu.VMEM((NCH, CHUNK), jnp.int32),
            pltpu.VMEM((NCH, CHUNK), jnp.int32),
            pltpu.VMEM((NBUF, CHUNK, 128), jnp.float32),
            pltpu.VMEM((ROWS_W // 2, N_CAT * D), jnp.float32),
            pltpu.SemaphoreType.DMA,
        ],
        compiler_params=pltpu.CompilerParams(
            use_tc_tiling_on_sc=True, needs_layout_passes=False),
    )
    def k(table_hbm, idx_hbm, bl_hbm, col_hbm, out_hbm,
          idx_v, sub_v, bl_v, col_v, raw_v, cmp_v, sem):
        wid = lax.axis_index("s") * NC + lax.axis_index("c")
        pltpu.sync_copy(idx_hbm.at[wid], idx_v)
        pltpu.sync_copy(bl_hbm, bl_v)
        pltpu.sync_copy(col_hbm, col_v)

        iota = lax.iota(jnp.int32, L)

        def transform(j, carry):
            for g in range(CHUNK // L):
                sl = pl.ds(g * L, L)
                v = idx_v[j, sl]
                sub_v[j, sl] = v & 7
                idx_v[j, sl] = v >> 3
            return carry

        lax.fori_loop(0, NCH, transform, 0)

        def fire(j, buf):
            pltpu.make_async_copy(
                table_hbm.at[idx_v.at[j]], raw_v.at[buf], sem,
            ).start()

        for b in range(NBUF):
            fire(b, b)

        def step(j, carry):
            buf = lax.rem(j, NBUF)
            pltpu.make_async_copy(
                table_hbm.at[idx_v.at[0]], raw_v.at[0], sem
            ).wait()
            bufv = jnp.full((L,), buf, jnp.int32)
            half = j // (NCH // 2)
            hoff = jnp.full((L,), half * (ROWS_W // 2), jnp.int32)
            for g in range(CHUNK // L):
                sl = pl.ds(g * L, L)
                svec = sub_v[j, sl] * D
                ivec = jnp.full((L,), g * L, jnp.int32) + iota
                bl = bl_v[j, sl] - hoff
                col0 = col_v[j, sl]
                for c in range(D):
                    cvec = jnp.full((L,), c, jnp.int32)
                    vals = plsc.load_gather(raw_v, [bufv, ivec, svec + cvec])
                    plsc.store_scatter(cmp_v, [bl, col0 + cvec], vals)

            @pl.when(j < NCH - NBUF)
            def _():
                fire(j + NBUF, buf)

            @pl.when(jnp.logical_or(j == NCH // 2 - 1, j == NCH - 1))
            def _():
                pltpu.sync_copy(
                    cmp_v,
                    out_hbm.at[pl.ds(wid * ROWS_W + half * (ROWS_W // 2),
                                     ROWS_W // 2)])

            return carry

        lax.fori_loop(0, NCH, step, 0)

    return k(table2, idx3, bl_tab, col_tab)


def _mlp_body(emb_ref, cont_ref, w10, b10, wo0, bo0, w11, b11, wo1, bo1,
              wout, bout, out_ref):
    x = jnp.concatenate([emb_ref[...], cont_ref[...]], axis=1)
    for (w1, b1, wo, bo) in ((w10, b10, wo0, bo0), (w11, b11, wo1, bo1)):
        h = jnp.maximum(
            jnp.dot(x, w1[...], preferred_element_type=jnp.float32) + b1[...],
            0.0)
        o = jnp.dot(h, wo[...], preferred_element_type=jnp.float32) + bo[...]
        x = jnp.maximum(o + x, 0.0)
    z = jnp.dot(x, wout[...], preferred_element_type=jnp.float32) + bout[...]
    out_ref[...] = jax.nn.sigmoid(z)


def _mlp(emb_flat, cont, w10, b10, wo0, bo0, w11, b11, wo1, bo1, wout, bout,
         blk=2048):
    grid = (B // blk,)
    full = lambda shape: pl.BlockSpec(shape, lambda i: (0, 0))
    return pl.pallas_call(
        _mlp_body,
        grid=grid,
        in_specs=[
            pl.BlockSpec((blk, N_CAT * D), lambda i: (i, 0)),
            pl.BlockSpec((blk, 3), lambda i: (i, 0)),
            full((D_IN, H)), full((1, H)), full((H, D_IN)), full((1, D_IN)),
            full((D_IN, H)), full((1, H)), full((H, D_IN)), full((1, D_IN)),
            full((D_IN, 1)), full((1, 1)),
        ],
        out_specs=pl.BlockSpec((blk, 1), lambda i: (i, 0)),
        out_shape=jax.ShapeDtypeStruct((B, 1), jnp.float32),
    )(emb_flat, cont, w10, b10, wo0, bo0, w11, b11, wo1, bo1, wout, bout)


def kernel(uid, iid, utag1, utag2, utag3, utag4, itag1, itag2, itag3, itag4,
           itag4_origin, itag4_square, itag4_cube,
           embed, W1_0, b1_0, Wo_0, bo_0, W1_1, b1_1, Wo_1, bo_1, Wout, bout):
    x_cate = jnp.concatenate(
        [uid, iid, utag1, utag2, utag3, utag4, itag1, itag2, itag3, itag4],
        axis=1)
    idx = x_cate.reshape(NW, NCH, CHUNK)
    table2 = embed.reshape(V // 8, 128)
    kl = jnp.arange(PER_W, dtype=jnp.int32).reshape(NCH, CHUNK)
    bl_tab = kl // N_CAT
    col_tab = (kl % N_CAT) * D
    emb_flat = _sc_gather(table2, idx, bl_tab, col_tab)
    cont = jnp.concatenate([itag4_origin, itag4_square, itag4_cube], axis=1)
    return _mlp(emb_flat, cont,
                W1_0, b1_0.reshape(1, H), Wo_0, bo_0.reshape(1, D_IN),
                W1_1, b1_1.reshape(1, H), Wo_1, bo_1.reshape(1, D_IN),
                Wout, bout.reshape(1, 1))


# speedup vs baseline: 1.0010x; 1.0010x over previous
import functools

import jax
import jax.numpy as jnp
from jax import lax
from jax.experimental import pallas as pl
from jax.experimental.pallas import tpu as pltpu
from jax.experimental.pallas import tpu_sc as plsc

B = 16384
V = 1000000
D = 16
N_CAT = 10
D_IN = N_CAT * D + 3
H = 32

NC = 2
NS = 16
NW = NC * NS
TOT = B * N_CAT
PER_W = TOT // NW
ROWS_W = B // NW
CHUNK = 128
NCH = PER_W // CHUNK
NBUF = 2
L = 16


def _sc_gather(table2, idx3, bl_tab, col_tab):
    mesh = plsc.VectorSubcoreMesh(core_axis_name="c", subcore_axis_name="s")

    @functools.partial(
        pl.kernel,
        out_type=jax.ShapeDtypeStruct((B, N_CAT * D), jnp.float32),
        mesh=mesh,
        scratch_types=[
            pltpu.VMEM((NCH, CHUNK), jnp.int32),
            pltpu.VMEM((NCH, CHUNK), jnp.int32),
            pltpu.VMEM((NCH, CHUNK), jnp.int32),
            pltpu.VMEM((NCH, CHUNK), jnp.int32),
            pltpu.VMEM((NBUF, CHUNK, 128), jnp.float32),
            pltpu.VMEM((ROWS_W // 2, N_CAT * D), jnp.float32),
            pltpu.SemaphoreType.DMA,
        ],
        compiler_params=pltpu.CompilerParams(
            use_tc_tiling_on_sc=True, needs_layout_passes=False),
    )
    def k(table_hbm, idx_hbm, bl_hbm, col_hbm, out_hbm,
          idx_v, sub_v, bl_v, col_v, raw_v, cmp_v, sem):
        wid = lax.axis_index("s") * NC + lax.axis_index("c")
        pltpu.sync_copy(idx_hbm.at[wid], idx_v)
        pltpu.sync_copy(bl_hbm, bl_v)
        pltpu.sync_copy(col_hbm, col_v)

        iota = lax.iota(jnp.int32, L)

        def transform(j, carry):
            for g in range(CHUNK // L):
                sl = pl.ds(g * L, L)
                v = idx_v[j, sl]
                sub_v[j, sl] = v & 7
                idx_v[j, sl] = v >> 3
            return carry

        lax.fori_loop(0, NCH, transform, 0)

        def fire(j, buf):
            pltpu.make_async_copy(
                table_hbm.at[idx_v.at[j]], raw_v.at[buf], sem,
            ).start()

        for b in range(NBUF):
            fire(b, b)

        def step(j, carry):
            buf = lax.rem(j, NBUF)
            pltpu.make_async_copy(
                table_hbm.at[idx_v.at[0]], raw_v.at[0], sem
            ).wait()
            bufv = jnp.full((L,), buf, jnp.int32)
            half = j // (NCH // 2)
            hoff = jnp.full((L,), half * (ROWS_W // 2), jnp.int32)
            for g in range(CHUNK // L):
                sl = pl.ds(g * L, L)
                svec = sub_v[j, sl] * D
                ivec = jnp.full((L,), g * L, jnp.int32) + iota
                bl = bl_v[j, sl] - hoff
                col0 = col_v[j, sl]
                for c in range(D):
                    cvec = jnp.full((L,), c, jnp.int32)
                    vals = plsc.load_gather(raw_v, [bufv, ivec, svec + cvec])
                    plsc.store_scatter(cmp_v, [bl, col0 + cvec], vals)

            @pl.when(j < NCH - NBUF)
            def _():
                fire(j + NBUF, buf)

            @pl.when(jnp.logical_or(j == NCH // 2 - 1, j == NCH - 1))
            def _():
                pltpu.sync_copy(
                    cmp_v,
                    out_hbm.at[pl.ds(wid * ROWS_W + half * (ROWS_W // 2),
                                     ROWS_W // 2)])

            return carry

        lax.fori_loop(0, NCH, step, 0)

    return k(table2, idx3, bl_tab, col_tab)


TBLK = 16384


def _transpose_body(x_ref, o_ref):
    x = x_ref[...]
    o_ref[...] = x.T.reshape(TBLK // 8, 128)


def _transpose_table(emb_t):
    grid = ((V + TBLK - 1) // TBLK,)
    return pl.pallas_call(
        _transpose_body,
        grid=grid,
        in_specs=[pl.BlockSpec((D, TBLK), lambda i: (0, i))],
        out_specs=pl.BlockSpec((TBLK // 8, 128), lambda i: (i, 0)),
        out_shape=jax.ShapeDtypeStruct((V // 8, 128), jnp.float32),
    )(emb_t)


def _mlp_body(emb_ref, cont_ref, w10, b10, wo0, bo0, w11, b11, wo1, bo1,
              wout, bout, out_ref):
    x = jnp.concatenate([emb_ref[...], cont_ref[...]], axis=1)
    for (w1, b1, wo, bo) in ((w10, b10, wo0, bo0), (w11, b11, wo1, bo1)):
        h = jnp.maximum(
            jnp.dot(x, w1[...], preferred_element_type=jnp.float32) + b1[...],
            0.0)
        o = jnp.dot(h, wo[...], preferred_element_type=jnp.float32) + bo[...]
        x = jnp.maximum(o + x, 0.0)
    z = jnp.dot(x, wout[...], preferred_element_type=jnp.float32) + bout[...]
    out_ref[...] = jax.nn.sigmoid(z)


def _mlp(emb_flat, cont, w10, b10, wo0, bo0, w11, b11, wo1, bo1, wout, bout,
         blk=2048):
    grid = (B // blk,)
    full = lambda shape: pl.BlockSpec(shape, lambda i: (0, 0))
    return pl.pallas_call(
        _mlp_body,
        grid=grid,
        in_specs=[
            pl.BlockSpec((blk, N_CAT * D), lambda i: (i, 0)),
            pl.BlockSpec((blk, 3), lambda i: (i, 0)),
            full((D_IN, H)), full((1, H)), full((H, D_IN)), full((1, D_IN)),
            full((D_IN, H)), full((1, H)), full((H, D_IN)), full((1, D_IN)),
            full((D_IN, 1)), full((1, 1)),
        ],
        out_specs=pl.BlockSpec((blk, 1), lambda i: (i, 0)),
        out_shape=jax.ShapeDtypeStruct((B, 1), jnp.float32),
    )(emb_flat, cont, w10, b10, wo0, bo0, w11, b11, wo1, bo1, wout, bout)


def kernel(uid, iid, utag1, utag2, utag3, utag4, itag1, itag2, itag3, itag4,
           itag4_origin, itag4_square, itag4_cube,
           embed, W1_0, b1_0, Wo_0, bo_0, W1_1, b1_1, Wo_1, bo_1, Wout, bout):
    x_cate = jnp.concatenate(
        [uid, iid, utag1, utag2, utag3, utag4, itag1, itag2, itag3, itag4],
        axis=1)
    idx = x_cate.reshape(NW, NCH, CHUNK)
    table2 = embed.reshape(V // 8, 128) + 0.0
    kl = jnp.arange(PER_W, dtype=jnp.int32).reshape(NCH, CHUNK)
    bl_tab = kl // N_CAT
    col_tab = (kl % N_CAT) * D
    emb_flat = _sc_gather(table2, idx, bl_tab, col_tab)
    cont = jnp.concatenate([itag4_origin, itag4_square, itag4_cube], axis=1)
    return _mlp(emb_flat, cont,
                W1_0, b1_0.reshape(1, H), Wo_0, bo_0.reshape(1, D_IN),
                W1_1, b1_1.reshape(1, H), Wo_1, bo_1.reshape(1, D_IN),
                Wout, bout.reshape(1, 1))
